# baseline (speedup 1.0000x reference)
import functools

import jax
import jax.numpy as jnp
from jax import lax
from jax.experimental import pallas as pl
from jax.experimental.pallas import tpu as pltpu
from jax.experimental.pallas import tpu_sc as plsc

_B, _VOC, _D, _K = 4096, 1000, 16, 20

_NW = 32
_BSC = 1536
_BTC = _B - _BSC
_NCG = 12
_NVR = 5
_VR = _VOC // _NVR

_BB = 256
_BB1 = 512


def _log_sigmoid(x):
    return jnp.minimum(x, 0.0) - jnp.log1p(jnp.exp(-jnp.abs(x)))


def _embed_body(viT_ref, voT_ref, VT_ref, UT_ref, w_ref, s1_ref):
    VT = VT_ref[...]
    UT = UT_ref[...]
    vi_eT = jnp.dot(VT, viT_ref[...], preferred_element_type=jnp.float32)
    vo_eT = jnp.dot(UT, voT_ref[...], preferred_element_type=jnp.float32)
    left = _log_sigmoid(jnp.sum(vi_eT * vo_eT, axis=0, keepdims=True))
    w_ref[...] = jax.lax.dot_general(UT, vi_eT, (((0,), (0,)), ((), ())),
                                     preferred_element_type=jnp.float32)

    @pl.when(pl.program_id(0) == 0)
    def _():
        s1_ref[0, 0] = 0.0

    s1_ref[0, 0] += jnp.sum(left)


def _run_embed(viT, voT, VT, UT):
    return pl.pallas_call(
        _embed_body,
        grid=(_BSC // _BB1,),
        in_specs=[
            pl.BlockSpec((_VOC, _BB1), lambda i: (0, i)),
            pl.BlockSpec((_VOC, _BB1), lambda i: (0, i)),
            pl.BlockSpec((_D, _VOC), lambda i: (0, 0)),
            pl.BlockSpec((_D, _VOC), lambda i: (0, 0)),
        ],
        out_specs=[
            pl.BlockSpec((_VOC, _BB1), lambda i: (0, i)),
            pl.BlockSpec(memory_space=pltpu.SMEM),
        ],
        out_shape=[
            jax.ShapeDtypeStruct((_VOC, _BSC), jnp.float32),
            jax.ShapeDtypeStruct((1, 1), jnp.float32),
        ],
    )(viT, voT, VT, UT)


def _sc_neg_copy(negT_hbm, nbuf, sem, k, vr0, col0, slot):
    return pltpu.make_async_copy(
        negT_hbm.at[k, pl.ds(vr0, _VR), pl.ds(col0, 128)],
        nbuf.at[slot],
        sem.at[slot],
    )


def _sc_body(negT_hbm, w_hbm, pbm_hbm, wbuf, nbuf, obuf, sem):
    c = lax.axis_index("c")
    s = lax.axis_index("s")
    wid = s * 2 + c
    for t in range(2):
        tid = wid * 2 + t

        @pl.when(tid < _NCG * _NVR)
        def _():
            col0 = pl.multiple_of(lax.rem(tid, _NCG) * 128, 128)
            vr_idx = tid // _NCG
            vr0 = pl.multiple_of(vr_idx * _VR, 8)
            pltpu.sync_copy(w_hbm.at[pl.ds(vr0, _VR), pl.ds(col0, 128)], wbuf)
            _sc_neg_copy(negT_hbm, nbuf, sem, 0, vr0, col0, 0).start()
            for k in range(_K):
                if k + 1 < _K:
                    _sc_neg_copy(negT_hbm, nbuf, sem, k + 1, vr0, col0,
                                 (k + 1) % 2).start()
                _sc_neg_copy(negT_hbm, nbuf, sem, k, vr0, col0, k % 2).wait()
                nb = nbuf.at[k % 2]

                def vstep(v, accs, nb=nb):
                    return tuple(
                        accs[j] + nb[v, pl.ds(16 * j, 16)] * wbuf[v, pl.ds(16 * j, 16)]
                        for j in range(8))

                accs = lax.fori_loop(
                    0, _VR, vstep,
                    tuple(jnp.zeros((16,), jnp.float32) for _ in range(8)),
                    unroll=2)
                for j in range(8):
                    obuf[k, pl.ds(16 * j, 16)] = accs[j]
            pltpu.sync_copy(obuf, pbm_hbm.at[vr_idx, :, pl.ds(col0, 128)])


def _run_sc(negT, W):
    mesh = plsc.VectorSubcoreMesh(core_axis_name="c", subcore_axis_name="s")
    kfn = pl.kernel(
        _sc_body,
        mesh=mesh,
        out_type=jax.ShapeDtypeStruct((_NVR, _K, _BSC), jnp.float32),
        scratch_types=[
            pltpu.VMEM((_VR, 128), jnp.float32),
            pltpu.VMEM((2, _VR, 128), jnp.float32),
            pltpu.VMEM((_K, 128), jnp.float32),
            pltpu.SemaphoreType.DMA((2,)),
        ],
    )
    return kfn(negT, W)


def _tc_body(viT_ref, voT_ref, negA_ref, negB_ref, VT_ref, UT_ref, out_ref):
    VT = VT_ref[...]
    UT = UT_ref[...]
    vi_eT = jnp.dot(VT, viT_ref[...], preferred_element_type=jnp.float32)
    vo_eT = jnp.dot(UT, voT_ref[...], preferred_element_type=jnp.float32)
    acc = _log_sigmoid(jnp.sum(vi_eT * vo_eT, axis=0, keepdims=True))
    for negT_ref in (negA_ref, negB_ref):
        for k in range(_K // 2):
            neT = jnp.dot(UT, negT_ref[k], preferred_element_type=jnp.float32)
            bm_k = jnp.sum(neT * vi_eT, axis=0, keepdims=True)
            acc = acc + _log_sigmoid(-bm_k)
    partial = -jnp.sum(acc) * (1.0 / _B)

    @pl.when(pl.program_id(0) == 0)
    def _():
        out_ref[0, 0] = 0.0

    out_ref[0, 0] += partial


def _run_tc(viT, voT, negT, VT, UT):
    off = _BSC // _BB
    return pl.pallas_call(
        _tc_body,
        grid=(_BTC // _BB,),
        in_specs=[
            pl.BlockSpec((_VOC, _BB), lambda i: (0, i + off)),
            pl.BlockSpec((_VOC, _BB), lambda i: (0, i + off)),
            pl.BlockSpec((_K // 2, _VOC, _BB), lambda i: (0, 0, i + off)),
            pl.BlockSpec((_K // 2, _VOC, _BB), lambda i: (1, 0, i + off)),
            pl.BlockSpec((_D, _VOC), lambda i: (0, 0)),
            pl.BlockSpec((_D, _VOC), lambda i: (0, 0)),
        ],
        out_specs=pl.BlockSpec(memory_space=pltpu.SMEM),
        out_shape=jax.ShapeDtypeStruct((1, 1), jnp.float32),
    )(viT, voT, negT, negT, VT, UT)


def _combine_body(pbm_ref, s1_ref, s2_ref, out_ref):
    bm = pbm_ref[0]
    for r in range(1, _NVR):
        bm = bm + pbm_ref[r]
    right = jnp.sum(_log_sigmoid(-bm))
    out_ref[0, 0] = s2_ref[0, 0] - (s1_ref[0, 0] + right) * (1.0 / _B)


def _run_combine(pbm, s1, s2):
    return pl.pallas_call(
        _combine_body,
        in_specs=[
            pl.BlockSpec((_NVR, _K, _BSC), lambda: (0, 0, 0)),
            pl.BlockSpec(memory_space=pltpu.SMEM),
            pl.BlockSpec(memory_space=pltpu.SMEM),
        ],
        out_specs=pl.BlockSpec(memory_space=pltpu.SMEM),
        out_shape=jax.ShapeDtypeStruct((1, 1), jnp.float32),
    )(pbm, s1, s2)


def kernel(vi, vo, neg_samples, V, U):
    viT = vi.T
    voT = vo.T
    negT = jnp.transpose(neg_samples, (1, 2, 0))
    VT = V.T
    UT = U.T

    W, s1 = _run_embed(viT, voT, VT, UT)
    pbm = _run_sc(negT, W)
    s2 = _run_tc(viT, voT, negT, VT, UT)
    out = _run_combine(pbm, s1, s2)
    return out[0, 0]


# speedup vs baseline: 1.2542x; 1.2542x over previous
import jax
import jax.numpy as jnp
from jax.experimental import pallas as pl
from jax.experimental.pallas import tpu as pltpu

_B, _VOC, _D, _K = 4096, 1000, 16, 20
_BB = 256


def _log_sigmoid(x):
    return jnp.minimum(x, 0.0) - jnp.log1p(jnp.exp(-jnp.abs(x)))


def _body(viT_ref, voT_ref, negA_ref, negB_ref, VT_ref, UT_ref, out_ref):
    VT = VT_ref[...]
    UT = UT_ref[...]
    vi_eT = jnp.dot(VT, viT_ref[...], preferred_element_type=jnp.float32)
    vo_eT = jnp.dot(UT, voT_ref[...], preferred_element_type=jnp.float32)
    acc = _log_sigmoid(jnp.sum(vi_eT * vo_eT, axis=0, keepdims=True))
    for negT_ref in (negA_ref, negB_ref):
        for k in range(_K // 2):
            neT = jnp.dot(UT, negT_ref[k], preferred_element_type=jnp.float32)
            bm_k = jnp.sum(neT * vi_eT, axis=0, keepdims=True)
            acc = acc + _log_sigmoid(-bm_k)
    partial = -jnp.sum(acc) * (1.0 / _B)

    @pl.when(pl.program_id(0) == 0)
    def _():
        out_ref[0, 0] = 0.0

    out_ref[0, 0] += partial


def kernel(vi, vo, neg_samples, V, U):
    viT = vi.T
    voT = vo.T
    negT = jnp.transpose(neg_samples, (1, 2, 0))
    VT = V.T
    UT = U.T
    out = pl.pallas_call(
        _body,
        grid=(_B // _BB,),
        in_specs=[
            pl.BlockSpec((_VOC, _BB), lambda i: (0, i)),
            pl.BlockSpec((_VOC, _BB), lambda i: (0, i)),
            pl.BlockSpec((_K // 2, _VOC, _BB), lambda i: (0, 0, i)),
            pl.BlockSpec((_K // 2, _VOC, _BB), lambda i: (1, 0, i)),
            pl.BlockSpec((_D, _VOC), lambda i: (0, 0)),
            pl.BlockSpec((_D, _VOC), lambda i: (0, 0)),
        ],
        out_specs=pl.BlockSpec(memory_space=pltpu.SMEM),
        out_shape=jax.ShapeDtypeStruct((1, 1), jnp.float32),
    )(viT, voT, negT, negT, VT, UT)
    return out[0, 0]
